# baseline (speedup 1.0000x reference)
import jax
import jax.numpy as jnp
from jax import lax
from jax.experimental import pallas as pl
from jax.experimental.pallas import tpu as pltpu
from jax.experimental.pallas import tpu_sc as plsc

_K = 16
_N = 32768
_ROWS = 2048
_L = 16
_NTILES = 32
_ROWS_PER = _ROWS // _NTILES
_GV = 16
_GE = _GV * _L
_NG = _N // _GE
_NCH = 8
_CAP = 512
_IMAX = 2147483647


def _isum(mask):
    return jnp.sum(mask.astype(jnp.int32))


def _popcnt(mask, cntv):
    cntv[pl.ds(0, _L)] = plsc.all_reduce_population_count(mask)
    return cntv[pl.ds(0, _L)][0]


def _row_pass(buf, hb, gmaxT, cvals, cidx, gvals, gidx, eidx, hitg, hitb,
              cnts, cntv, iota, i16s, i128, ninf):
    @plsc.parallel_loop(0, _NG, unroll=8, carry=(ninf, ninf))
    def _pa(g, carry):
        m1, m2 = carry
        base = hb + g * _GE
        a0 = buf[pl.ds(base, _L)]
        a1 = buf[pl.ds(base + _L, _L)]
        a2 = buf[pl.ds(base + 2 * _L, _L)]
        a3 = buf[pl.ds(base + 3 * _L, _L)]
        for t in range(1, _GV // 4):
            a0 = jnp.maximum(a0, buf[pl.ds(base + (4 * t) * _L, _L)])
            a1 = jnp.maximum(a1, buf[pl.ds(base + (4 * t + 1) * _L, _L)])
            a2 = jnp.maximum(a2, buf[pl.ds(base + (4 * t + 2) * _L, _L)])
            a3 = jnp.maximum(a3, buf[pl.ds(base + (4 * t + 3) * _L, _L)])
        gm = jnp.maximum(jnp.maximum(a0, a1), jnp.maximum(a2, a3))
        plsc.store_scatter(gmaxT, [i128 + g], gm)
        return (jnp.maximum(m1, gm),
                jnp.maximum(m2, jnp.minimum(m1, gm)))

    m1, m2 = _pa
    s1, _u = plsc.sort_key_val(m1, iota, descending=True)
    s2 = jnp.sort(m2)
    T = jnp.min(jnp.maximum(s1, s2))
    Tv = jnp.full((_L,), T, jnp.float32)

    cnts[0] = 0
    for s in range(_NCH):
        g0 = gmaxT[pl.ds(s * _L, _L)]
        g1 = gmaxT[pl.ds(128 + s * _L, _L)]
        g2 = gmaxT[pl.ds(256 + s * _L, _L)]
        g3 = gmaxT[pl.ds(384 + s * _L, _L)]
        for b in range(4, _L):
            r = b & 3
            if r == 0:
                g0 = jnp.maximum(g0, gmaxT[pl.ds(b * 128 + s * _L, _L)])
            elif r == 1:
                g1 = jnp.maximum(g1, gmaxT[pl.ds(b * 128 + s * _L, _L)])
            elif r == 2:
                g2 = jnp.maximum(g2, gmaxT[pl.ds(b * 128 + s * _L, _L)])
            else:
                g3 = jnp.maximum(g3, gmaxT[pl.ds(b * 128 + s * _L, _L)])
        gh = jnp.maximum(jnp.maximum(g0, g1), jnp.maximum(g2, g3))
        hmask = gh >= Tv
        c = cnts[0]
        plsc.store_compressed(hitg.at[pl.ds(c, _L)], s * _L + iota, mask=hmask)
        cnts[0] = c + _popcnt(hmask, cntv)
    nhg = cnts[0]

    cnts[1] = 0

    def _b2(j, _):
        g = hitg[pl.ds(j, _L)][0]
        gm = plsc.load_gather(gmaxT, [i128 + g])
        bmask = gm >= Tv
        c = cnts[1]
        w = jnp.minimum(c, _CAP)
        plsc.store_compressed(hitb.at[pl.ds(w, _L)], g * _L + iota, mask=bmask)
        cnts[1] = c + _popcnt(bmask, cntv)
        return 0

    lax.fori_loop(0, nhg, _b2, 0)
    nb = jnp.minimum(cnts[1], _CAP)

    cnts[2] = 0

    def _b3(j, _):
        bid = hitb[pl.ds(j, _L)][0]
        gi = (bid >> 4) * _GE + (bid & (_L - 1)) + i16s
        v = plsc.load_gather(buf, [gi + hb])
        sel = v >= Tv
        c = cnts[2]
        w = jnp.minimum(c, _CAP)
        plsc.store_compressed(cvals.at[pl.ds(w, _L)], v, mask=sel)
        plsc.store_compressed(cidx.at[pl.ds(w, _L)], gi, mask=sel)
        cnts[2] = c + _popcnt(sel, cntv)
        return 0

    lax.fori_loop(0, nb, _b3, 0)
    nc = jnp.minimum(cnts[2], _CAP)
    cvals[pl.ds(nc, _L)] = ninf

    rd, _u = plsc.sort_key_val(cvals[pl.ds(0, _L)], iota, descending=True)
    nv = (nc + _L - 1) // _L

    def _c1(i, r):
        ca = jnp.sort(cvals[pl.ds(i * _L, _L)])
        rr, _u2 = plsc.sort_key_val(jnp.maximum(r, ca), iota, descending=True)
        return rr

    rd = lax.fori_loop(1, nv, _c1, rd)
    t = jnp.min(rd)
    tv = jnp.full((_L,), t, jnp.float32)
    m = _isum(rd > tv)

    cnts[1] = 0
    cnts[2] = 0

    def _c2(i, _):
        v = cvals[pl.ds(i * _L, _L)]
        ix = cidx[pl.ds(i * _L, _L)]
        gt = v > tv
        ngt = cnts[1]
        plsc.store_compressed(gvals.at[pl.ds(ngt, _L)], v, mask=gt)
        plsc.store_compressed(gidx.at[pl.ds(ngt, _L)], ix, mask=gt)
        cnts[1] = ngt + _popcnt(gt, cntv)
        eq = v == tv
        neq = cnts[2]
        w = jnp.minimum(neq, _CAP)
        plsc.store_compressed(eidx.at[pl.ds(w, _L)], ix, mask=eq)
        cnts[2] = neq + _popcnt(eq, cntv)
        return 0

    lax.fori_loop(0, nv, _c2, 0)
    neq = jnp.minimum(cnts[2], _CAP)
    eidx[pl.ds(neq, _L)] = jnp.full((_L,), _IMAX, jnp.int32)

    e0 = jnp.sort(eidx[pl.ds(0, _L)])
    nve = (neq + _L - 1) // _L

    def _ce(i, e):
        cd, _u3 = plsc.sort_key_val(eidx[pl.ds(i * _L, _L)], iota,
                                    descending=True)
        return jnp.sort(jnp.minimum(e, cd))

    e0 = lax.fori_loop(1, nve, _ce, e0)

    eidx[pl.ds(0, _L)] = e0
    esh = plsc.load_gather(eidx, [jnp.maximum(iota - m, 0)])
    fin_i = jnp.where(iota < m, gidx[pl.ds(0, _L)], esh)
    fin_v = jnp.where(iota < m, gvals[pl.ds(0, _L)], tv)
    _sk, sv = plsc.sort_key_val(fin_i, fin_v, descending=False)
    return sv


def _sc_body(x_hbm, out_hbm, buf, gmaxT, cvals, cidx, gvals, gidx, eidx,
             hitg, hitb, ostage, cnts, cntv, sem0, sem1):
    wid = lax.axis_index("s") * 2 + lax.axis_index("c")
    row0 = wid * _ROWS_PER
    iota = lax.iota(jnp.int32, _L)
    i16s = iota * _L
    i128 = iota * _NG
    ninf = jnp.full((_L,), -jnp.inf, jnp.float32)

    def src(r):
        rr = row0 + r
        return x_hbm.at[rr // 32, rr % 32]

    pltpu.async_copy(src(0), buf.at[pl.ds(0, _N)], sem0)

    def row_pair(rr, _):
        r0 = 2 * rr
        pltpu.make_async_copy(src(r0), buf.at[pl.ds(0, _N)], sem0).wait()
        pltpu.async_copy(src(r0 + 1), buf.at[pl.ds(_N, _N)], sem1)
        out0 = _row_pass(buf, 0, gmaxT, cvals, cidx, gvals, gidx, eidx,
                         hitg, hitb, cnts, cntv, iota, i16s, i128, ninf)
        ostage[pl.ds(r0 * _K, _K)] = out0
        pltpu.make_async_copy(src(r0 + 1), buf.at[pl.ds(_N, _N)], sem1).wait()

        @pl.when(rr < _ROWS_PER // 2 - 1)
        def _():
            pltpu.async_copy(src(r0 + 2), buf.at[pl.ds(0, _N)], sem0)

        out1 = _row_pass(buf, _N, gmaxT, cvals, cidx, gvals, gidx, eidx,
                         hitg, hitb, cnts, cntv, iota, i16s, i128, ninf)
        ostage[pl.ds((r0 + 1) * _K, _K)] = out1
        return 0

    lax.fori_loop(0, _ROWS_PER // 2, row_pair, 0)
    pltpu.sync_copy(ostage, out_hbm.at[pl.ds(row0 * _K, _ROWS_PER * _K)])


@jax.jit
def _kmax_sc(x):
    mesh = plsc.VectorSubcoreMesh(core_axis_name="c", subcore_axis_name="s")
    f = pl.kernel(
        _sc_body,
        out_type=jax.ShapeDtypeStruct((_ROWS * _K,), jnp.float32),
        mesh=mesh,
        compiler_params=pltpu.CompilerParams(needs_layout_passes=False),
        scratch_types=[
            pltpu.VMEM((2 * _N,), jnp.float32),
            pltpu.VMEM((_NG * _L,), jnp.float32),
            pltpu.VMEM((_CAP + 2 * _L,), jnp.float32),
            pltpu.VMEM((_CAP + 2 * _L,), jnp.int32),
            pltpu.VMEM((2 * _L,), jnp.float32),
            pltpu.VMEM((2 * _L,), jnp.int32),
            pltpu.VMEM((_CAP + 2 * _L,), jnp.int32),
            pltpu.VMEM((_NG + _L,), jnp.int32),
            pltpu.VMEM((_CAP + 2 * _L,), jnp.int32),
            pltpu.VMEM((_ROWS_PER * _K,), jnp.float32),
            pltpu.SMEM((8,), jnp.int32),
            pltpu.VMEM((_L,), jnp.int32),
            pltpu.SemaphoreType.DMA,
            pltpu.SemaphoreType.DMA,
        ],
    )
    return f(x)


def kernel(inputs):
    B, C, N = inputs.shape
    out = _kmax_sc(inputs)
    return out.reshape(B, C, _K)


# speedup vs baseline: 1.0619x; 1.0619x over previous
import jax
import jax.numpy as jnp
from jax import lax
from jax.experimental import pallas as pl
from jax.experimental.pallas import tpu as pltpu
from jax.experimental.pallas import tpu_sc as plsc

_K = 16
_N = 32768
_ROWS = 2048
_L = 16
_NTILES = 32
_ROWS_PER = _ROWS // _NTILES
_GV = 16
_GE = _GV * _L
_NG = _N // _GE
_NCH = 8
_CAP = 512
_IMAX = 2147483647


def _isum(mask):
    return jnp.sum(mask.astype(jnp.int32))


def _popcnt(mask, cntv):
    cntv[pl.ds(0, _L)] = plsc.all_reduce_population_count(mask)
    return cntv[pl.ds(0, _L)][0]


def _row_pass(buf, hb, gmaxT, cvals, cidx, gvals, gidx, eidx, hitg, hitb,
              cnts, cntv, iota, i16s, i128, ninf):
    @plsc.parallel_loop(0, _NG, unroll=4, carry=(ninf, ninf))
    def _pa(g, carry):
        m1, m2 = carry
        base = hb + g * _GE
        a0 = buf[pl.ds(base, _L)]
        a1 = buf[pl.ds(base + _L, _L)]
        a2 = buf[pl.ds(base + 2 * _L, _L)]
        a3 = buf[pl.ds(base + 3 * _L, _L)]
        for t in range(1, _GV // 4):
            a0 = jnp.maximum(a0, buf[pl.ds(base + (4 * t) * _L, _L)])
            a1 = jnp.maximum(a1, buf[pl.ds(base + (4 * t + 1) * _L, _L)])
            a2 = jnp.maximum(a2, buf[pl.ds(base + (4 * t + 2) * _L, _L)])
            a3 = jnp.maximum(a3, buf[pl.ds(base + (4 * t + 3) * _L, _L)])
        gm = jnp.maximum(jnp.maximum(a0, a1), jnp.maximum(a2, a3))
        plsc.store_scatter(gmaxT, [i128 + g], gm)
        return (jnp.maximum(m1, gm),
                jnp.maximum(m2, jnp.minimum(m1, gm)))

    m1, m2 = _pa
    s1, _u = plsc.sort_key_val(m1, iota, descending=True)
    s2 = jnp.sort(m2)
    T = jnp.min(jnp.maximum(s1, s2))
    Tv = jnp.full((_L,), T, jnp.float32)

    cnts[0] = 0
    for s in range(_NCH):
        g0 = gmaxT[pl.ds(s * _L, _L)]
        g1 = gmaxT[pl.ds(128 + s * _L, _L)]
        g2 = gmaxT[pl.ds(256 + s * _L, _L)]
        g3 = gmaxT[pl.ds(384 + s * _L, _L)]
        for b in range(4, _L):
            r = b & 3
            if r == 0:
                g0 = jnp.maximum(g0, gmaxT[pl.ds(b * 128 + s * _L, _L)])
            elif r == 1:
                g1 = jnp.maximum(g1, gmaxT[pl.ds(b * 128 + s * _L, _L)])
            elif r == 2:
                g2 = jnp.maximum(g2, gmaxT[pl.ds(b * 128 + s * _L, _L)])
            else:
                g3 = jnp.maximum(g3, gmaxT[pl.ds(b * 128 + s * _L, _L)])
        gh = jnp.maximum(jnp.maximum(g0, g1), jnp.maximum(g2, g3))
        hmask = gh >= Tv
        c = cnts[0]
        plsc.store_compressed(hitg.at[pl.ds(c, _L)], s * _L + iota, mask=hmask)
        cnts[0] = c + _popcnt(hmask, cntv)
    nhg = cnts[0]

    cnts[1] = 0

    def _b2(j, _):
        g = hitg[pl.ds(j, _L)][0]
        gm = plsc.load_gather(gmaxT, [i128 + g])
        bmask = gm >= Tv
        c = cnts[1]
        w = jnp.minimum(c, _CAP)
        plsc.store_compressed(hitb.at[pl.ds(w, _L)], g * _L + iota, mask=bmask)
        cnts[1] = c + _popcnt(bmask, cntv)
        return 0

    lax.fori_loop(0, nhg, _b2, 0)
    nb = jnp.minimum(cnts[1], _CAP)

    cnts[2] = 0

    def _b3(j, _):
        bid = hitb[pl.ds(j, _L)][0]
        gi = (bid >> 4) * _GE + (bid & (_L - 1)) + i16s
        v = plsc.load_gather(buf, [gi + hb])
        sel = v >= Tv
        c = cnts[2]
        w = jnp.minimum(c, _CAP)
        plsc.store_compressed(cvals.at[pl.ds(w, _L)], v, mask=sel)
        plsc.store_compressed(cidx.at[pl.ds(w, _L)], gi, mask=sel)
        cnts[2] = c + _popcnt(sel, cntv)
        return 0

    lax.fori_loop(0, nb, _b3, 0)
    nc = jnp.minimum(cnts[2], _CAP)
    cvals[pl.ds(nc, _L)] = ninf

    rd, _u = plsc.sort_key_val(cvals[pl.ds(0, _L)], iota, descending=True)
    nv = (nc + _L - 1) // _L

    def _c1(i, r):
        ca = jnp.sort(cvals[pl.ds(i * _L, _L)])
        rr, _u2 = plsc.sort_key_val(jnp.maximum(r, ca), iota, descending=True)
        return rr

    rd = lax.fori_loop(1, nv, _c1, rd)
    t = jnp.min(rd)
    tv = jnp.full((_L,), t, jnp.float32)
    m = _isum(rd > tv)

    cnts[1] = 0
    cnts[2] = 0

    def _c2(i, _):
        v = cvals[pl.ds(i * _L, _L)]
        ix = cidx[pl.ds(i * _L, _L)]
        gt = v > tv
        ngt = cnts[1]
        plsc.store_compressed(gvals.at[pl.ds(ngt, _L)], v, mask=gt)
        plsc.store_compressed(gidx.at[pl.ds(ngt, _L)], ix, mask=gt)
        cnts[1] = ngt + _popcnt(gt, cntv)
        eq = v == tv
        neq = cnts[2]
        w = jnp.minimum(neq, _CAP)
        plsc.store_compressed(eidx.at[pl.ds(w, _L)], ix, mask=eq)
        cnts[2] = neq + _popcnt(eq, cntv)
        return 0

    lax.fori_loop(0, nv, _c2, 0)
    neq = jnp.minimum(cnts[2], _CAP)
    eidx[pl.ds(neq, _L)] = jnp.full((_L,), _IMAX, jnp.int32)

    e0 = jnp.sort(eidx[pl.ds(0, _L)])
    nve = (neq + _L - 1) // _L

    def _ce(i, e):
        cd, _u3 = plsc.sort_key_val(eidx[pl.ds(i * _L, _L)], iota,
                                    descending=True)
        return jnp.sort(jnp.minimum(e, cd))

    e0 = lax.fori_loop(1, nve, _ce, e0)

    eidx[pl.ds(0, _L)] = e0
    esh = plsc.load_gather(eidx, [jnp.maximum(iota - m, 0)])
    fin_i = jnp.where(iota < m, gidx[pl.ds(0, _L)], esh)
    fin_v = jnp.where(iota < m, gvals[pl.ds(0, _L)], tv)
    _sk, sv = plsc.sort_key_val(fin_i, fin_v, descending=False)
    return sv


def _sc_body(x_hbm, out_hbm, buf, gmaxT, cvals, cidx, gvals, gidx, eidx,
             hitg, hitb, ostage, cnts, cntv, sem0, sem1):
    wid = lax.axis_index("s") * 2 + lax.axis_index("c")
    row0 = wid * _ROWS_PER
    iota = lax.iota(jnp.int32, _L)
    i16s = iota * _L
    i128 = iota * _NG
    ninf = jnp.full((_L,), -jnp.inf, jnp.float32)

    def src(r):
        rr = row0 + r
        return x_hbm.at[rr // 32, rr % 32]

    pltpu.async_copy(src(0), buf.at[pl.ds(0, _N)], sem0)

    def row_pair(rr, _):
        r0 = 2 * rr
        pltpu.make_async_copy(src(r0), buf.at[pl.ds(0, _N)], sem0).wait()
        pltpu.async_copy(src(r0 + 1), buf.at[pl.ds(_N, _N)], sem1)
        out0 = _row_pass(buf, 0, gmaxT, cvals, cidx, gvals, gidx, eidx,
                         hitg, hitb, cnts, cntv, iota, i16s, i128, ninf)
        ostage[pl.ds(r0 * _K, _K)] = out0
        pltpu.make_async_copy(src(r0 + 1), buf.at[pl.ds(_N, _N)], sem1).wait()

        @pl.when(rr < _ROWS_PER // 2 - 1)
        def _():
            pltpu.async_copy(src(r0 + 2), buf.at[pl.ds(0, _N)], sem0)

        out1 = _row_pass(buf, _N, gmaxT, cvals, cidx, gvals, gidx, eidx,
                         hitg, hitb, cnts, cntv, iota, i16s, i128, ninf)
        ostage[pl.ds((r0 + 1) * _K, _K)] = out1
        return 0

    lax.fori_loop(0, _ROWS_PER // 2, row_pair, 0)
    pltpu.sync_copy(ostage, out_hbm.at[pl.ds(row0 * _K, _ROWS_PER * _K)])


@jax.jit
def _kmax_sc(x):
    mesh = plsc.VectorSubcoreMesh(core_axis_name="c", subcore_axis_name="s")
    f = pl.kernel(
        _sc_body,
        out_type=jax.ShapeDtypeStruct((_ROWS * _K,), jnp.float32),
        mesh=mesh,
        compiler_params=pltpu.CompilerParams(needs_layout_passes=False),
        scratch_types=[
            pltpu.VMEM((2 * _N,), jnp.float32),
            pltpu.VMEM((_NG * _L,), jnp.float32),
            pltpu.VMEM((_CAP + 2 * _L,), jnp.float32),
            pltpu.VMEM((_CAP + 2 * _L,), jnp.int32),
            pltpu.VMEM((2 * _L,), jnp.float32),
            pltpu.VMEM((2 * _L,), jnp.int32),
            pltpu.VMEM((_CAP + 2 * _L,), jnp.int32),
            pltpu.VMEM((_NG + _L,), jnp.int32),
            pltpu.VMEM((_CAP + 2 * _L,), jnp.int32),
            pltpu.VMEM((_ROWS_PER * _K,), jnp.float32),
            pltpu.SMEM((8,), jnp.int32),
            pltpu.VMEM((_L,), jnp.int32),
            pltpu.SemaphoreType.DMA,
            pltpu.SemaphoreType.DMA,
        ],
    )
    return f(x)


def kernel(inputs):
    B, C, N = inputs.shape
    out = _kmax_sc(inputs)
    return out.reshape(B, C, _K)
